# Initial kernel scaffold; baseline (speedup 1.0000x reference)
import jax
import jax.numpy as jnp
from jax.experimental import pallas as pl


def kernel(input, table0, table1, table2):
    raise NotImplementedError("write your pallas kernel here")


# speedup vs baseline: 41.6855x; 41.6855x over previous
import functools

import numpy as np
import jax
import jax.numpy as jnp
from jax import lax
from jax.experimental import pallas as pl
from jax.experimental.pallas import tpu as pltpu
from jax.experimental.pallas import tpu_sc as plsc

N = 1048576
NLVL = 16
F = 2
T = 1 << 19
PRIME1 = np.uint32(2654435761)
NC, NS, L = 2, 16, 16
NW = NC * NS
PPW = N // NW
C = 64
G = C // L
NCH = PPW // C
NSLOT = 3 * NLVL

_RES = [int(np.floor(16 * (1.5 ** l))) for l in range(NLVL)]

@functools.cache
def _build_kernel():
    mesh = plsc.VectorSubcoreMesh(
        core_axis_name="c", subcore_axis_name="s",
        num_cores=NC, num_subcores=NS)
    return functools.partial(
        pl.kernel,
        out_type=jax.ShapeDtypeStruct((N, 2 * NLVL), jnp.float32),
        mesh=mesh,
        scratch_types=[
            pltpu.VMEM((3, C), jnp.float32),
            pltpu.VMEM((NLVL,), jnp.float32),
            pltpu.VMEM((NSLOT, 4, C), jnp.int32),
            pltpu.VMEM((NSLOT, 4, C, 8), jnp.float32),
            pltpu.VMEM((C, 2 * NLVL), jnp.float32),
            pltpu.SemaphoreType.DMA((NLVL,)),
        ],
        compiler_params=pltpu.CompilerParams(
            needs_layout_passes=False, use_tc_tiling_on_sc=False),
    )(_tri_body)


def _tri_body(in0, in1, in2, tab0, tab1, tab2, resf, out,
                coords_v, consts_v, idx_v, rows_v, out_v, sem):
    wid = lax.axis_index("s") * NC + lax.axis_index("c")
    base0 = wid * PPW
    pltpu.sync_copy(resf, consts_v)
    iota = lax.iota(jnp.int32, L)
    zeros = jnp.zeros((L,), jnp.int32)
    tabs = (tab0, tab1, tab2)

    def corners(x, y, res_spl, s_i, dense, lofs):
        cx = (x * res_spl).astype(jnp.int32)
        cy = (y * res_spl).astype(jnp.int32)
        out_idx = []
        for dx, dy in ((0, 0), (0, 1), (1, 0), (1, 1)):
            a = cx + dx
            b = cy + dy
            idx_d = a * s_i + b
            h = a.astype(jnp.uint32) ^ (b.astype(jnp.uint32) * PRIME1)
            idx_h = (h & np.uint32(T - 1)).astype(jnp.int32)
            out_idx.append(jnp.where(dense, idx_d, idx_h) + lofs)
        return out_idx

    def chunk_body(ch, carry):
        pt = base0 + ch * C
        for p, inp in enumerate((in0, in1, in2)):
            pltpu.sync_copy(inp.at[pl.ds(pt, C)], coords_v.at[p])

        def fire_level(l, carry2):
            res_spl = plsc.load_gather(consts_v, [zeros + l])
            s_i = res_spl.astype(jnp.int32) + 1
            dense = (s_i * s_i) <= T
            lofs = zeros + l * T
            for p in range(3):
                k = l * 3 + p

                def grp(g, carry3):
                    col = g * L + iota
                    x = plsc.load_gather(coords_v, [zeros + p, col])
                    y = plsc.load_gather(coords_v, [zeros + ((p + 1) % 3), col])
                    idx4 = corners(x, y, res_spl, s_i, dense, lofs)
                    for c in range(4):
                        plsc.store_scatter(
                            idx_v, [zeros + k, zeros + c, col], idx4[c] >> 2)
                    return carry3

                lax.fori_loop(0, G, grp, 0)
                for c in range(4):
                    pltpu.async_copy(
                        tabs[p].at[idx_v.at[k, c]], rows_v.at[k, c], sem.at[l])
            return carry2

        lax.fori_loop(0, NLVL, fire_level, 0)

        def interp_level(l, carry2):
            res_spl = plsc.load_gather(consts_v, [zeros + l])
            for p in range(3):
                k = l * 3 + p
                for c in range(4):
                    pltpu.make_async_copy(
                        tabs[p].at[idx_v.at[k, c]], rows_v.at[k, c],
                        sem.at[l]).wait()

            def grp(g, carry3):
                col = g * L + iota
                accs = []
                for p in range(3):
                    k = l * 3 + p
                    x = plsc.load_gather(coords_v, [zeros + p, col])
                    y = plsc.load_gather(coords_v, [zeros + ((p + 1) % 3), col])
                    px = x * res_spl
                    py = y * res_spl
                    fx = px - px.astype(jnp.int32).astype(jnp.float32)
                    fy = py - py.astype(jnp.int32).astype(jnp.float32)
                    s_i = res_spl.astype(jnp.int32) + 1
                    dense = (s_i * s_i) <= T
                    idx4 = corners(x, y, res_spl, s_i, dense, zeros)
                    offs = [(idx4[c] & 3) * 2 for c in range(4)]
                    w = ((1.0 - fx) * (1.0 - fy), (1.0 - fx) * fy,
                         fx * (1.0 - fy), fx * fy)
                    acc = []
                    for ff in range(F):
                        s = None
                        for c in range(4):
                            v = plsc.load_gather(
                                rows_v, [zeros + k, zeros + c, col, offs[c] + ff])
                            s = w[c] * v if s is None else s + w[c] * v
                        acc.append(s)
                    accs.append(acc)
                for ff in range(F):
                    o = jnp.maximum(accs[0][ff] * accs[1][ff], 1e-6)
                    o = jnp.maximum(o * accs[2][ff], 1e-6)
                    plsc.store_scatter(out_v, [col, zeros + (2 * l + ff)], o)
                return carry3

            lax.fori_loop(0, G, grp, 0)
            return carry2

        lax.fori_loop(0, NLVL, interp_level, 0)
        pltpu.sync_copy(out_v, out.at[pl.ds(pt, C)])
        return carry

    lax.fori_loop(0, NCH, chunk_body, 0)


def kernel(input, table0, table1, table2):
    in0, in1, in2 = input[:, 0], input[:, 1], input[:, 2]
    resf = jnp.array(_RES, jnp.float32)
    t0 = table0.reshape(NLVL * T * F // 8, 8)
    t1 = table1.reshape(NLVL * T * F // 8, 8)
    t2 = table2.reshape(NLVL * T * F // 8, 8)
    return _build_kernel()(in0, in1, in2, t0, t1, t2, resf)


# speedup vs baseline: 129.7194x; 3.1119x over previous
import functools

import numpy as np
import jax
import jax.numpy as jnp
from jax import lax
from jax.experimental import pallas as pl
from jax.experimental.pallas import tpu as pltpu
from jax.experimental.pallas import tpu_sc as plsc

N = 1048576
NLVL = 16
F = 2
T = 1 << 19
PRIME1 = np.uint32(2654435761)
NC, NS, L = 2, 16, 16
NW = NC * NS
PPW = N // NW
C = 64
G = C // L
NCH = PPW // C
NSLOT = 3 * NLVL

_RES = [int(np.floor(16 * (1.5 ** l))) for l in range(NLVL)]

NBLK = NLVL * T * F // 8
NB = 64
RIT = NLVL * (T // 128) // NB // NW


@functools.cache
def _build_relayout():
    mesh = plsc.VectorSubcoreMesh(
        core_axis_name="c", subcore_axis_name="s",
        num_cores=NC, num_subcores=NS)
    ot = jax.ShapeDtypeStruct((NBLK, 8), jnp.float32)
    return functools.partial(
        pl.kernel,
        out_type=(ot, ot, ot),
        mesh=mesh,
        scratch_types=[
            pltpu.VMEM((NB, 2, 128), jnp.float32),
            pltpu.VMEM((32 * NB, 8), jnp.float32),
        ],
        compiler_params=pltpu.CompilerParams(
            needs_layout_passes=False, use_tc_tiling_on_sc=False),
    )(_relayout_body)


def _relayout_body(tv0, tv1, tv2, o0, o1, o2, nat_v, int_v):
    wid = lax.axis_index("s") * NC + lax.axis_index("c")
    iota = lax.iota(jnp.int32, L)
    zeros = jnp.zeros((L,), jnp.int32)
    for p, (tv, o) in enumerate(((tv0, o0), (tv1, o1), (tv2, o2))):
        def step(i, carry):
            gid = wid * RIT + i
            l = gid // (4096 // NB)
            j = gid % (4096 // NB)
            tb0 = j * NB
            pltpu.sync_copy(tv.at[l, pl.ds(tb0, NB)], nat_v)

            def blk(b, carry2):
                for g in range(8):
                    col = g * L + iota
                    off = 256 * b + 32 * g + 2 * iota
                    for f in range(2):
                        v = plsc.load_gather(nat_v, [zeros + b, zeros + f, col])
                        plsc.store_scatter(
                            int_v, [(off + f) >> 3, (off + f) & 7], v)
                return carry2

            lax.fori_loop(0, NB, blk, 0)
            r0 = l * (T * F // 8) + 32 * tb0
            pltpu.sync_copy(int_v, o.at[pl.ds(r0, 32 * NB)])
            return carry

        lax.fori_loop(0, RIT, step, 0)


@functools.cache
def _build_kernel():
    mesh = plsc.VectorSubcoreMesh(
        core_axis_name="c", subcore_axis_name="s",
        num_cores=NC, num_subcores=NS)
    return functools.partial(
        pl.kernel,
        out_type=jax.ShapeDtypeStruct((4, N // 128, 8, 128), jnp.float32),
        mesh=mesh,
        scratch_types=[
            pltpu.VMEM((3, C), jnp.float32),
            pltpu.VMEM((NLVL,), jnp.float32),
            pltpu.VMEM((NSLOT, 4, C), jnp.int32),
            pltpu.VMEM((NSLOT, 4, C, 8), jnp.float32),
            pltpu.VMEM((4, 8, 128), jnp.float32),
            pltpu.SemaphoreType.DMA((NLVL,)),
        ],
        compiler_params=pltpu.CompilerParams(
            needs_layout_passes=False, use_tc_tiling_on_sc=False),
    )(_tri_body)


def _tri_body(in0, in1, in2, tab0, tab1, tab2, resf, out,
                coords_v, consts_v, idx_v, rows_v, out_v, sem):
    wid = lax.axis_index("s") * NC + lax.axis_index("c")
    base0 = wid * PPW
    pltpu.sync_copy(resf, consts_v)
    iota = lax.iota(jnp.int32, L)
    zeros = jnp.zeros((L,), jnp.int32)
    tabs = (tab0, tab1, tab2)

    def corners(x, y, res_spl, s_i, dense, lofs):
        cx = (x * res_spl).astype(jnp.int32)
        cy = (y * res_spl).astype(jnp.int32)
        out_idx = []
        for dx, dy in ((0, 0), (0, 1), (1, 0), (1, 1)):
            a = cx + dx
            b = cy + dy
            idx_d = a * s_i + b
            h = a.astype(jnp.uint32) ^ (b.astype(jnp.uint32) * PRIME1)
            idx_h = (h & np.uint32(T - 1)).astype(jnp.int32)
            out_idx.append(jnp.where(dense, idx_d, idx_h) + lofs)
        return out_idx

    def chunk_body(ch, carry):
      for half in range(2):
        pt = base0 + ch * (2 * C) + half * C
        for p, inp in enumerate((in0, in1, in2)):
            pltpu.sync_copy(inp.at[pl.ds(pt, C)], coords_v.at[p])

        def fire_level(l, carry2):
            res_spl = plsc.load_gather(consts_v, [zeros + l])
            s_i = res_spl.astype(jnp.int32) + 1
            dense = (s_i * s_i) <= T
            lofs = zeros + l * T
            for p in range(3):
                k = l * 3 + p

                def grp(g, carry3):
                    col = g * L + iota
                    x = plsc.load_gather(coords_v, [zeros + p, col])
                    y = plsc.load_gather(coords_v, [zeros + ((p + 1) % 3), col])
                    idx4 = corners(x, y, res_spl, s_i, dense, lofs)
                    for c in range(4):
                        plsc.store_scatter(
                            idx_v, [zeros + k, zeros + c, col], idx4[c] >> 2)
                    return carry3

                lax.fori_loop(0, G, grp, 0)
                for c in range(4):
                    pltpu.async_copy(
                        tabs[p].at[idx_v.at[k, c]], rows_v.at[k, c], sem.at[l])
            return carry2

        lax.fori_loop(0, NLVL, fire_level, 0)

        def interp_level(l, carry2):
            res_spl = plsc.load_gather(consts_v, [zeros + l])
            for p in range(3):
                k = l * 3 + p
                for c in range(4):
                    pltpu.make_async_copy(
                        tabs[p].at[idx_v.at[k, c]], rows_v.at[k, c],
                        sem.at[l]).wait()

            def grp(g, carry3):
                col = g * L + iota
                accs = []
                for p in range(3):
                    k = l * 3 + p
                    x = plsc.load_gather(coords_v, [zeros + p, col])
                    y = plsc.load_gather(coords_v, [zeros + ((p + 1) % 3), col])
                    px = x * res_spl
                    py = y * res_spl
                    fx = px - px.astype(jnp.int32).astype(jnp.float32)
                    fy = py - py.astype(jnp.int32).astype(jnp.float32)
                    s_i = res_spl.astype(jnp.int32) + 1
                    dense = (s_i * s_i) <= T
                    idx4 = corners(x, y, res_spl, s_i, dense, zeros)
                    offs = [(idx4[c] & 3) * 2 for c in range(4)]
                    w = ((1.0 - fx) * (1.0 - fy), (1.0 - fx) * fy,
                         fx * (1.0 - fy), fx * fy)
                    acc = []
                    for ff in range(F):
                        s = None
                        for c in range(4):
                            v = plsc.load_gather(
                                rows_v, [zeros + k, zeros + c, col, offs[c] + ff])
                            s = w[c] * v if s is None else s + w[c] * v
                        acc.append(s)
                    accs.append(acc)
                for ff in range(F):
                    o = jnp.maximum(accs[0][ff] * accs[1][ff], 1e-6)
                    o = jnp.maximum(o * accs[2][ff], 1e-6)
                    fcol = 2 * l + ff
                    plsc.store_scatter(
                        out_v,
                        [zeros + (fcol >> 3), zeros + (fcol & 7),
                         half * C + col], o)
                return carry3

            lax.fori_loop(0, G, grp, 0)
            return carry2

        lax.fori_loop(0, NLVL, interp_level, 0)
      pltpu.sync_copy(out_v, out.at[:, base0 // 128 + ch])
      return carry

    lax.fori_loop(0, NCH // 2, chunk_body, 0)


def kernel(input, table0, table1, table2):
    in0, in1, in2 = input[:, 0], input[:, 1], input[:, 2]
    resf = jnp.array(_RES, jnp.float32)

    def view(t):
        return t.reshape(NLVL, T // 128, 128, F).transpose(0, 1, 3, 2)

    t0, t1, t2 = _build_relayout()(view(table0), view(table1), view(table2))
    out4 = _build_kernel()(in0, in1, in2, t0, t1, t2, resf)
    return out4.transpose(1, 3, 0, 2).reshape(N, 2 * NLVL)


# speedup vs baseline: 227.2260x; 1.7517x over previous
import functools

import numpy as np
import jax
import jax.numpy as jnp
from jax import lax
from jax.experimental import pallas as pl
from jax.experimental.pallas import tpu as pltpu
from jax.experimental.pallas import tpu_sc as plsc

N = 1048576
NLVL = 16
F = 2
T = 1 << 19
PRIME1 = np.uint32(2654435761)
NC, NS, L = 2, 16, 16
NW = NC * NS
PPW = N // NW
C = 128
G = C // L
NCH = PPW // C
D = 4
NDENSE = 10

_RES = [int(np.floor(16 * (1.5 ** l))) for l in range(NLVL)]
assert sum((r + 1) * (r + 1) <= T for r in _RES) == NDENSE

NBLK = NLVL * T * F // 8
LBLK = T * F // 8
NB = 64
RIT = NLVL * (T // 128) // NB // NW


@functools.cache
def _build_relayout():
    mesh = plsc.VectorSubcoreMesh(
        core_axis_name="c", subcore_axis_name="s",
        num_cores=NC, num_subcores=NS)
    rm = jax.ShapeDtypeStruct((NBLK, 8), jnp.float32)
    pr = jax.ShapeDtypeStruct((NBLK + 1, 16), jnp.float32)
    return functools.partial(
        pl.kernel,
        out_type=(rm, rm, rm, pr, pr, pr),
        mesh=mesh,
        scratch_types=[
            pltpu.VMEM((NB, 2, 128), jnp.float32),
            pltpu.VMEM((32 * NB, 8), jnp.float32),
        ],
        compiler_params=pltpu.CompilerParams(
            needs_layout_passes=False, use_tc_tiling_on_sc=False),
    )(_relayout_body)


def _relayout_body(tv0, tv1, tv2, o0, o1, o2, p0, p1, p2, nat_v, int_v):
    wid = lax.axis_index("s") * NC + lax.axis_index("c")
    iota = lax.iota(jnp.int32, L)
    zeros = jnp.zeros((L,), jnp.int32)
    nq = 32 * NB
    for tv, o, pp in ((tv0, o0, p0), (tv1, o1, p1), (tv2, o2, p2)):
        def step(i, carry):
            gid = wid * RIT + i
            l = gid // (4096 // NB)
            j = gid % (4096 // NB)
            tb0 = j * NB
            pltpu.sync_copy(tv.at[l, pl.ds(tb0, NB)], nat_v)

            def blk(b, carry2):
                for g in range(8):
                    col = g * L + iota
                    off = 256 * b + 32 * g + 2 * iota
                    for f in range(2):
                        v = plsc.load_gather(nat_v, [zeros + b, zeros + f, col])
                        plsc.store_scatter(
                            int_v, [(off + f) >> 3, (off + f) & 7], v)
                return carry2

            lax.fori_loop(0, NB, blk, 0)
            q0 = l * LBLK + 32 * tb0
            pltpu.sync_copy(int_v, o.at[pl.ds(q0, nq)])
            pltpu.sync_copy(int_v, pp.at[pl.ds(q0 + 1, nq), pl.ds(0, 8)])
            pltpu.sync_copy(int_v, pp.at[pl.ds(q0, nq), pl.ds(8, 8)])
            return carry

        lax.fori_loop(0, RIT, step, 0)


@functools.cache
def _build_kernel():
    mesh = plsc.VectorSubcoreMesh(
        core_axis_name="c", subcore_axis_name="s",
        num_cores=NC, num_subcores=NS)
    return functools.partial(
        pl.kernel,
        out_type=jax.ShapeDtypeStruct((4, N // 128, 8, 128), jnp.float32),
        mesh=mesh,
        scratch_types=[
            pltpu.VMEM((3, C), jnp.float32),
            pltpu.VMEM((D, 3, 2, C), jnp.int32),
            pltpu.VMEM((D, 3, 4, C), jnp.int32),
            pltpu.VMEM((D, 3, 2, C, 16), jnp.float32),
            pltpu.VMEM((D, 3, 4, C, 8), jnp.float32),
            pltpu.VMEM((4, 8, 128), jnp.float32),
            pltpu.SemaphoreType.DMA((D,)),
        ],
        compiler_params=pltpu.CompilerParams(
            needs_layout_passes=False, use_tc_tiling_on_sc=False),
    )(_tri_body)


def _tri_body(in0, in1, in2, trm0, trm1, trm2, pt0, pt1, pt2, out,
              coords_v, idxd_v, idxh_v, dring, hring, out_v, sem):
    wid = lax.axis_index("s") * NC + lax.axis_index("c")
    base0 = wid * PPW
    iota = lax.iota(jnp.int32, L)
    zeros = jnp.zeros((L,), jnp.int32)
    trms = (trm0, trm1, trm2)
    pts = (pt0, pt1, pt2)

    def cells(p, col, res_f):
        x = plsc.load_gather(coords_v, [zeros + p, col])
        y = plsc.load_gather(coords_v, [zeros + ((p + 1) % 3), col])
        px = x * res_f
        py = y * res_f
        cx = px.astype(jnp.int32)
        cy = py.astype(jnp.int32)
        return px, py, cx, cy

    def hash_idx(a, b):
        h = a.astype(jnp.uint32) ^ (b.astype(jnp.uint32) * PRIME1)
        return (h & np.uint32(T - 1)).astype(jnp.int32)

    def fire(l, is_dense):
        res_f = float(_RES[l])
        s_i = _RES[l] + 1
        d = l % D
        lq = l * LBLK
        for p in range(3):
            def grp(g, carry):
                col = g * L + iota
                _, _, cx, cy = cells(p, col, res_f)
                if is_dense:
                    for j in range(2):
                        t0 = (cx + j) * s_i + cy
                        plsc.store_scatter(
                            idxd_v, [zeros + d, zeros + p, zeros + j, col],
                            (t0 >> 2) + (lq + 1))
                else:
                    for c, (dx, dy) in enumerate(
                            ((0, 0), (0, 1), (1, 0), (1, 1))):
                        t = hash_idx(cx + dx, cy + dy)
                        plsc.store_scatter(
                            idxh_v, [zeros + d, zeros + p, zeros + c, col],
                            (t >> 2) + lq)
                return carry

            lax.fori_loop(0, G, grp, 0)
            if is_dense:
                for j in range(2):
                    pltpu.async_copy(
                        pts[p].at[idxd_v.at[d, p, j]], dring.at[d, p, j],
                        sem.at[d])
            else:
                for c in range(4):
                    pltpu.async_copy(
                        trms[p].at[idxh_v.at[d, p, c]], hring.at[d, p, c],
                        sem.at[d])

    def interp(l, is_dense):
        res_f = float(_RES[l])
        s_i = _RES[l] + 1
        d = l % D
        for p in range(3):
            if is_dense:
                for j in range(2):
                    pltpu.make_async_copy(
                        pts[p].at[idxd_v.at[d, p, j]], dring.at[d, p, j],
                        sem.at[d]).wait()
            else:
                for c in range(4):
                    pltpu.make_async_copy(
                        trms[p].at[idxh_v.at[d, p, c]], hring.at[d, p, c],
                        sem.at[d]).wait()

        def grp(g, carry):
            col = g * L + iota
            accs = []
            for p in range(3):
                px, py, cx, cy = cells(p, col, res_f)
                fx = px - cx.astype(jnp.float32)
                fy = py - cy.astype(jnp.float32)
                wx = (1.0 - fx, fx)
                wy = (1.0 - fy, fy)
                acc = [None, None]
                if is_dense:
                    for j in range(2):
                        t0 = (cx + j) * s_i + cy
                        o0 = (t0 & 3) * 2
                        for ff in range(F):
                            v0 = plsc.load_gather(
                                dring,
                                [zeros + d, zeros + p, zeros + j, col,
                                 o0 + ff])
                            v1 = plsc.load_gather(
                                dring,
                                [zeros + d, zeros + p, zeros + j, col,
                                 o0 + (2 + ff)])
                            s = wx[j] * (wy[0] * v0 + wy[1] * v1)
                            acc[ff] = s if acc[ff] is None else acc[ff] + s
                else:
                    for c, (dx, dy) in enumerate(
                            ((0, 0), (0, 1), (1, 0), (1, 1))):
                        t = hash_idx(cx + dx, cy + dy)
                        ot = (t & 3) * 2
                        w = wx[dx] * wy[dy]
                        for ff in range(F):
                            v = plsc.load_gather(
                                hring,
                                [zeros + d, zeros + p, zeros + c, col,
                                 ot + ff])
                            s = w * v
                            acc[ff] = s if acc[ff] is None else acc[ff] + s
                accs.append(acc)
            for ff in range(F):
                o = jnp.maximum(accs[0][ff] * accs[1][ff], 1e-6)
                o = jnp.maximum(o * accs[2][ff], 1e-6)
                fcol = 2 * l + ff
                plsc.store_scatter(
                    out_v, [zeros + (fcol >> 3), zeros + (fcol & 7), col], o)
            return carry

        lax.fori_loop(0, G, grp, 0)

    def chunk_body(ch, carry):
        pt = base0 + ch * C
        for p, inp in enumerate((in0, in1, in2)):
            pltpu.sync_copy(inp.at[pl.ds(pt, C)], coords_v.at[p])
        for l in range(D):
            fire(l, l < NDENSE)
        for l in range(NLVL):
            interp(l, l < NDENSE)
            if l + D < NLVL:
                fire(l + D, (l + D) < NDENSE)
        pltpu.sync_copy(out_v, out.at[:, base0 // 128 + ch])
        return carry

    lax.fori_loop(0, NCH, chunk_body, 0)


def kernel(input, table0, table1, table2):
    in0, in1, in2 = input[:, 0], input[:, 1], input[:, 2]

    def view(t):
        return t.reshape(NLVL, T // 128, 128, F).transpose(0, 1, 3, 2)

    t0, t1, t2, p0, p1, p2 = _build_relayout()(
        view(table0), view(table1), view(table2))
    out4 = _build_kernel()(in0, in1, in2, t0, t1, t2, p0, p1, p2)
    return out4.transpose(1, 3, 0, 2).reshape(N, 2 * NLVL)


# speedup vs baseline: 234.0656x; 1.0301x over previous
import functools

import numpy as np
import jax
import jax.numpy as jnp
from jax import lax
from jax.experimental import pallas as pl
from jax.experimental.pallas import tpu as pltpu
from jax.experimental.pallas import tpu_sc as plsc

N = 1048576
NLVL = 16
F = 2
T = 1 << 19
PRIME1 = np.uint32(2654435761)
NC, NS, L = 2, 16, 16
NW = NC * NS
PPW = N // NW
C = 128
G = C // L
NCH = PPW // C
D = 4
NDENSE = 10

_RES = [int(np.floor(16 * (1.5 ** l))) for l in range(NLVL)]
assert sum((r + 1) * (r + 1) <= T for r in _RES) == NDENSE

NBLK = NLVL * T * F // 8
LBLK = T * F // 8
NB = 64
RIT = NLVL * (T // 128) // NB // NW


@functools.cache
def _build_relayout():
    mesh = plsc.VectorSubcoreMesh(
        core_axis_name="c", subcore_axis_name="s",
        num_cores=NC, num_subcores=NS)
    rm = jax.ShapeDtypeStruct((NBLK, 8), jnp.float32)
    pr = jax.ShapeDtypeStruct((NBLK + 1, 16), jnp.float32)
    return functools.partial(
        pl.kernel,
        out_type=(rm, rm, rm, pr, pr, pr),
        mesh=mesh,
        scratch_types=[
            pltpu.VMEM((NB, 2, 128), jnp.float32),
            pltpu.VMEM((32 * NB, 8), jnp.float32),
        ],
        compiler_params=pltpu.CompilerParams(
            needs_layout_passes=False, use_tc_tiling_on_sc=False),
    )(_relayout_body)


def _relayout_body(tv0, tv1, tv2, o0, o1, o2, p0, p1, p2, nat_v, int_v):
    wid = lax.axis_index("s") * NC + lax.axis_index("c")
    iota = lax.iota(jnp.int32, L)
    zeros = jnp.zeros((L,), jnp.int32)
    nq = 32 * NB
    spl = 4096 // NB
    dstep = NDENSE * spl // NW
    hstep = (NLVL - NDENSE) * spl // NW
    for tv, o, pp in ((tv0, o0, p0), (tv1, o1, p1), (tv2, o2, p2)):
        def interleave(gid):
            l = gid // spl
            tb0 = (gid % spl) * NB
            pltpu.sync_copy(tv.at[l, pl.ds(tb0, NB)], nat_v)

            def blk(b, carry2):
                for g in range(8):
                    col = g * L + iota
                    off = 256 * b + 32 * g + 2 * iota
                    for f in range(2):
                        v = plsc.load_gather(nat_v, [zeros + b, zeros + f, col])
                        plsc.store_scatter(
                            int_v, [(off + f) >> 3, (off + f) & 7], v)
                return carry2

            lax.fori_loop(0, NB, blk, 0)
            return l * LBLK + 32 * tb0

        def step_d(i, carry):
            q0 = interleave(wid * dstep + i)
            pltpu.sync_copy(int_v, pp.at[pl.ds(q0 + 1, nq), pl.ds(0, 8)])
            pltpu.sync_copy(int_v, pp.at[pl.ds(q0, nq), pl.ds(8, 8)])
            return carry

        def step_h(i, carry):
            q0 = interleave(NDENSE * spl + wid * hstep + i)
            pltpu.sync_copy(int_v, o.at[pl.ds(q0, nq)])
            return carry

        lax.fori_loop(0, dstep, step_d, 0)
        lax.fori_loop(0, hstep, step_h, 0)


@functools.cache
def _build_kernel():
    mesh = plsc.VectorSubcoreMesh(
        core_axis_name="c", subcore_axis_name="s",
        num_cores=NC, num_subcores=NS)
    return functools.partial(
        pl.kernel,
        out_type=jax.ShapeDtypeStruct((4, N // 128, 8, 128), jnp.float32),
        mesh=mesh,
        scratch_types=[
            pltpu.VMEM((3, C), jnp.float32),
            pltpu.VMEM((D, 3, 2, C), jnp.int32),
            pltpu.VMEM((D, 3, 4, C), jnp.int32),
            pltpu.VMEM((D, 3, 2, C, 16), jnp.float32),
            pltpu.VMEM((D, 3, 4, C, 8), jnp.float32),
            pltpu.VMEM((4, 8, 128), jnp.float32),
            pltpu.SemaphoreType.DMA((D,)),
        ],
        compiler_params=pltpu.CompilerParams(
            needs_layout_passes=False, use_tc_tiling_on_sc=False),
    )(_tri_body)


def _tri_body(in0, in1, in2, trm0, trm1, trm2, pt0, pt1, pt2, out,
              coords_v, idxd_v, idxh_v, dring, hring, out_v, sem):
    wid = lax.axis_index("s") * NC + lax.axis_index("c")
    base0 = wid * PPW
    iota = lax.iota(jnp.int32, L)
    zeros = jnp.zeros((L,), jnp.int32)
    trms = (trm0, trm1, trm2)
    pts = (pt0, pt1, pt2)

    def cells(p, col, res_f):
        x = plsc.load_gather(coords_v, [zeros + p, col])
        y = plsc.load_gather(coords_v, [zeros + ((p + 1) % 3), col])
        px = x * res_f
        py = y * res_f
        cx = px.astype(jnp.int32)
        cy = py.astype(jnp.int32)
        return px, py, cx, cy

    def hash_idx(a, b):
        h = a.astype(jnp.uint32) ^ (b.astype(jnp.uint32) * PRIME1)
        return (h & np.uint32(T - 1)).astype(jnp.int32)

    def fire(l, is_dense):
        res_f = float(_RES[l])
        s_i = _RES[l] + 1
        d = l % D
        lq = l * LBLK
        for p in range(3):
            def grp(g, carry):
                col = g * L + iota
                _, _, cx, cy = cells(p, col, res_f)
                if is_dense:
                    for j in range(2):
                        t0 = (cx + j) * s_i + cy
                        plsc.store_scatter(
                            idxd_v, [zeros + d, zeros + p, zeros + j, col],
                            (t0 >> 2) + (lq + 1))
                else:
                    for c, (dx, dy) in enumerate(
                            ((0, 0), (0, 1), (1, 0), (1, 1))):
                        t = hash_idx(cx + dx, cy + dy)
                        plsc.store_scatter(
                            idxh_v, [zeros + d, zeros + p, zeros + c, col],
                            (t >> 2) + lq)
                return carry

            lax.fori_loop(0, G, grp, 0)
            if is_dense:
                for j in range(2):
                    pltpu.async_copy(
                        pts[p].at[idxd_v.at[d, p, j]], dring.at[d, p, j],
                        sem.at[d])
            else:
                for c in range(4):
                    pltpu.async_copy(
                        trms[p].at[idxh_v.at[d, p, c]], hring.at[d, p, c],
                        sem.at[d])

    def interp(l, is_dense):
        res_f = float(_RES[l])
        s_i = _RES[l] + 1
        d = l % D
        for p in range(3):
            if is_dense:
                for j in range(2):
                    pltpu.make_async_copy(
                        pts[p].at[idxd_v.at[d, p, j]], dring.at[d, p, j],
                        sem.at[d]).wait()
            else:
                for c in range(4):
                    pltpu.make_async_copy(
                        trms[p].at[idxh_v.at[d, p, c]], hring.at[d, p, c],
                        sem.at[d]).wait()

        def grp(g, carry):
            col = g * L + iota
            accs = []
            for p in range(3):
                px, py, cx, cy = cells(p, col, res_f)
                fx = px - cx.astype(jnp.float32)
                fy = py - cy.astype(jnp.float32)
                wx = (1.0 - fx, fx)
                wy = (1.0 - fy, fy)
                acc = [None, None]
                if is_dense:
                    for j in range(2):
                        t0 = (cx + j) * s_i + cy
                        o0 = (t0 & 3) * 2
                        for ff in range(F):
                            v0 = plsc.load_gather(
                                dring,
                                [zeros + d, zeros + p, zeros + j, col,
                                 o0 + ff])
                            v1 = plsc.load_gather(
                                dring,
                                [zeros + d, zeros + p, zeros + j, col,
                                 o0 + (2 + ff)])
                            s = wx[j] * (wy[0] * v0 + wy[1] * v1)
                            acc[ff] = s if acc[ff] is None else acc[ff] + s
                else:
                    for c, (dx, dy) in enumerate(
                            ((0, 0), (0, 1), (1, 0), (1, 1))):
                        t = hash_idx(cx + dx, cy + dy)
                        ot = (t & 3) * 2
                        w = wx[dx] * wy[dy]
                        for ff in range(F):
                            v = plsc.load_gather(
                                hring,
                                [zeros + d, zeros + p, zeros + c, col,
                                 ot + ff])
                            s = w * v
                            acc[ff] = s if acc[ff] is None else acc[ff] + s
                accs.append(acc)
            for ff in range(F):
                o = jnp.maximum(accs[0][ff] * accs[1][ff], 1e-6)
                o = jnp.maximum(o * accs[2][ff], 1e-6)
                fcol = 2 * l + ff
                plsc.store_scatter(
                    out_v, [zeros + (fcol >> 3), zeros + (fcol & 7), col], o)
            return carry

        lax.fori_loop(0, G, grp, 0)

    def chunk_body(ch, carry):
        pt = base0 + ch * C
        for p, inp in enumerate((in0, in1, in2)):
            pltpu.sync_copy(inp.at[pl.ds(pt, C)], coords_v.at[p])
        for l in range(D):
            fire(l, l < NDENSE)
        for l in range(NLVL):
            interp(l, l < NDENSE)
            if l + D < NLVL:
                fire(l + D, (l + D) < NDENSE)
        pltpu.sync_copy(out_v, out.at[:, base0 // 128 + ch])
        return carry

    lax.fori_loop(0, NCH, chunk_body, 0)


def kernel(input, table0, table1, table2):
    in0, in1, in2 = input[:, 0], input[:, 1], input[:, 2]

    def view(t):
        return t.reshape(NLVL, T // 128, 128, F).transpose(0, 1, 3, 2)

    t0, t1, t2, p0, p1, p2 = _build_relayout()(
        view(table0), view(table1), view(table2))
    out4 = _build_kernel()(in0, in1, in2, t0, t1, t2, p0, p1, p2)
    return out4.transpose(1, 3, 0, 2).reshape(N, 2 * NLVL)


# speedup vs baseline: 294.2825x; 1.2573x over previous
import functools

import numpy as np
import jax
import jax.numpy as jnp
from jax import lax
from jax.experimental import pallas as pl
from jax.experimental.pallas import tpu as pltpu
from jax.experimental.pallas import tpu_sc as plsc

N = 1048576
NLVL = 16
F = 2
T = 1 << 19
PRIME1 = np.uint32(2654435761)
NC, NS, L = 2, 16, 16
NW = NC * NS
PPW = N // NW
C = 128
G = C // L
NCH = PPW // C
D = 4
NDENSE = 10

_RES = [int(np.floor(16 * (1.5 ** l))) for l in range(NLVL)]
assert sum((r + 1) * (r + 1) <= T for r in _RES) == NDENSE

NBLK = NLVL * T * F // 8
LBLK = T * F // 8
NST = 3
_ST_NB = [((r + 1) * (r + 1) * F + 7) // 8 for r in _RES[:NST]]
_ST_OFF = [sum(_ST_NB[:i]) for i in range(NST)]
ST_TOT = sum(_ST_NB)
NB = 64
RIT = NLVL * (T // 128) // NB // NW


@functools.cache
def _build_relayout():
    mesh = plsc.VectorSubcoreMesh(
        core_axis_name="c", subcore_axis_name="s",
        num_cores=NC, num_subcores=NS)
    rm = jax.ShapeDtypeStruct((NBLK, 8), jnp.float32)
    pr = jax.ShapeDtypeStruct((NBLK + 1, 16), jnp.float32)
    return functools.partial(
        pl.kernel,
        out_type=(rm, rm, rm, pr, pr, pr),
        mesh=mesh,
        scratch_types=[
            pltpu.VMEM((NB, 2, 128), jnp.float32),
            pltpu.VMEM((32 * NB, 8), jnp.float32),
        ],
        compiler_params=pltpu.CompilerParams(
            needs_layout_passes=False, use_tc_tiling_on_sc=False),
    )(_relayout_body)


def _relayout_body(tv0, tv1, tv2, o0, o1, o2, p0, p1, p2, nat_v, int_v):
    wid = lax.axis_index("s") * NC + lax.axis_index("c")
    iota = lax.iota(jnp.int32, L)
    zeros = jnp.zeros((L,), jnp.int32)
    nq = 32 * NB
    spl = 4096 // NB
    dstep = NDENSE * spl // NW
    hstep = (NLVL - NDENSE) * spl // NW
    for tv, o, pp in ((tv0, o0, p0), (tv1, o1, p1), (tv2, o2, p2)):
        def interleave(gid):
            l = gid // spl
            tb0 = (gid % spl) * NB
            pltpu.sync_copy(tv.at[l, pl.ds(tb0, NB)], nat_v)

            def blk(b, carry2):
                for g in range(8):
                    col = g * L + iota
                    off = 256 * b + 32 * g + 2 * iota
                    for f in range(2):
                        v = plsc.load_gather(nat_v, [zeros + b, zeros + f, col])
                        plsc.store_scatter(
                            int_v, [(off + f) >> 3, (off + f) & 7], v)
                return carry2

            lax.fori_loop(0, NB, blk, 0)
            return l * LBLK + 32 * tb0

        def step_d(i, carry):
            q0 = interleave(wid * dstep + i)
            pltpu.sync_copy(int_v, pp.at[pl.ds(q0 + 1, nq), pl.ds(0, 8)])
            pltpu.sync_copy(int_v, pp.at[pl.ds(q0, nq), pl.ds(8, 8)])
            return carry

        def step_h(i, carry):
            q0 = interleave(NDENSE * spl + wid * hstep + i)
            pltpu.sync_copy(int_v, o.at[pl.ds(q0, nq)])
            return carry

        lax.fori_loop(0, dstep, step_d, 0)
        lax.fori_loop(0, hstep, step_h, 0)


@functools.cache
def _build_kernel():
    mesh = plsc.VectorSubcoreMesh(
        core_axis_name="c", subcore_axis_name="s",
        num_cores=NC, num_subcores=NS)
    return functools.partial(
        pl.kernel,
        out_type=jax.ShapeDtypeStruct((4, N // 128, 8, 128), jnp.float32),
        mesh=mesh,
        scratch_types=[
            pltpu.VMEM((3, C), jnp.float32),
            pltpu.VMEM((D, 3, 2, C), jnp.int32),
            pltpu.VMEM((D, 3, 4, C), jnp.int32),
            pltpu.VMEM((D, 3, 2, C, 16), jnp.float32),
            pltpu.VMEM((D, 3, 4, C, 8), jnp.float32),
            pltpu.VMEM((4, 8, 128), jnp.float32),
            pltpu.VMEM((3, ST_TOT, 8), jnp.float32),
            pltpu.SemaphoreType.DMA((D,)),
        ],
        compiler_params=pltpu.CompilerParams(
            needs_layout_passes=False, use_tc_tiling_on_sc=False),
    )(_tri_body)


def _tri_body(in0, in1, in2, trm0, trm1, trm2, pt0, pt1, pt2, out,
              coords_v, idxd_v, idxh_v, dring, hring, out_v, stage_v, sem):
    wid = lax.axis_index("s") * NC + lax.axis_index("c")
    base0 = wid * PPW
    iota = lax.iota(jnp.int32, L)
    zeros = jnp.zeros((L,), jnp.int32)
    trms = (trm0, trm1, trm2)
    pts = (pt0, pt1, pt2)
    for p in range(3):
        for li in range(NST):
            pltpu.sync_copy(
                pts[p].at[pl.ds(li * LBLK, _ST_NB[li]), pl.ds(8, 8)],
                stage_v.at[p, pl.ds(_ST_OFF[li], _ST_NB[li])])

    def cells(p, col, res_f):
        x = plsc.load_gather(coords_v, [zeros + p, col])
        y = plsc.load_gather(coords_v, [zeros + ((p + 1) % 3), col])
        px = x * res_f
        py = y * res_f
        cx = px.astype(jnp.int32)
        cy = py.astype(jnp.int32)
        return px, py, cx, cy

    def hash_idx(a, b):
        h = a.astype(jnp.uint32) ^ (b.astype(jnp.uint32) * PRIME1)
        return (h & np.uint32(T - 1)).astype(jnp.int32)

    def fire(l, is_dense):
        res_f = float(_RES[l])
        s_i = _RES[l] + 1
        d = l % D
        lq = l * LBLK
        for p in range(3):
            def grp(g, carry):
                col = g * L + iota
                _, _, cx, cy = cells(p, col, res_f)
                if is_dense:
                    for j in range(2):
                        t0 = (cx + j) * s_i + cy
                        plsc.store_scatter(
                            idxd_v, [zeros + d, zeros + p, zeros + j, col],
                            (t0 >> 2) + (lq + 1))
                else:
                    for c, (dx, dy) in enumerate(
                            ((0, 0), (0, 1), (1, 0), (1, 1))):
                        t = hash_idx(cx + dx, cy + dy)
                        plsc.store_scatter(
                            idxh_v, [zeros + d, zeros + p, zeros + c, col],
                            (t >> 2) + lq)
                return carry

            lax.fori_loop(0, G, grp, 0)
            if is_dense:
                for j in range(2):
                    pltpu.async_copy(
                        pts[p].at[idxd_v.at[d, p, j]], dring.at[d, p, j],
                        sem.at[d])
            else:
                for c in range(4):
                    pltpu.async_copy(
                        trms[p].at[idxh_v.at[d, p, c]], hring.at[d, p, c],
                        sem.at[d])

    def interp(l, is_dense):
        res_f = float(_RES[l])
        s_i = _RES[l] + 1
        d = l % D
        for p in range(3):
            if is_dense:
                for j in range(2):
                    pltpu.make_async_copy(
                        pts[p].at[idxd_v.at[d, p, j]], dring.at[d, p, j],
                        sem.at[d]).wait()
            else:
                for c in range(4):
                    pltpu.make_async_copy(
                        trms[p].at[idxh_v.at[d, p, c]], hring.at[d, p, c],
                        sem.at[d]).wait()

        def grp(g, carry):
            col = g * L + iota
            accs = []
            for p in range(3):
                px, py, cx, cy = cells(p, col, res_f)
                fx = px - cx.astype(jnp.float32)
                fy = py - cy.astype(jnp.float32)
                wx = (1.0 - fx, fx)
                wy = (1.0 - fy, fy)
                acc = [None, None]
                if is_dense:
                    for j in range(2):
                        t0 = (cx + j) * s_i + cy
                        o0 = (t0 & 3) * 2
                        for ff in range(F):
                            v0 = plsc.load_gather(
                                dring,
                                [zeros + d, zeros + p, zeros + j, col,
                                 o0 + ff])
                            v1 = plsc.load_gather(
                                dring,
                                [zeros + d, zeros + p, zeros + j, col,
                                 o0 + (2 + ff)])
                            s = wx[j] * (wy[0] * v0 + wy[1] * v1)
                            acc[ff] = s if acc[ff] is None else acc[ff] + s
                else:
                    for c, (dx, dy) in enumerate(
                            ((0, 0), (0, 1), (1, 0), (1, 1))):
                        t = hash_idx(cx + dx, cy + dy)
                        ot = (t & 3) * 2
                        w = wx[dx] * wy[dy]
                        for ff in range(F):
                            v = plsc.load_gather(
                                hring,
                                [zeros + d, zeros + p, zeros + c, col,
                                 ot + ff])
                            s = w * v
                            acc[ff] = s if acc[ff] is None else acc[ff] + s
                accs.append(acc)
            for ff in range(F):
                o = jnp.maximum(accs[0][ff] * accs[1][ff], 1e-6)
                o = jnp.maximum(o * accs[2][ff], 1e-6)
                fcol = 2 * l + ff
                plsc.store_scatter(
                    out_v, [zeros + (fcol >> 3), zeros + (fcol & 7), col], o)
            return carry

        lax.fori_loop(0, G, grp, 0)

    def interp_staged(l):
        res_f = float(_RES[l])
        s_i = _RES[l] + 1
        woff = _ST_OFF[l] * 8

        def grp(g, carry):
            col = g * L + iota
            accs = []
            for p in range(3):
                px, py, cx, cy = cells(p, col, res_f)
                fx = px - cx.astype(jnp.float32)
                fy = py - cy.astype(jnp.float32)
                wx = (1.0 - fx, fx)
                wy = (1.0 - fy, fy)
                acc = [None, None]
                for j in range(2):
                    t0 = (cx + j) * s_i + cy
                    w0 = woff + t0 * 2
                    for ff in range(F):
                        v0 = plsc.load_gather(
                            stage_v,
                            [zeros + p, (w0 + ff) >> 3, (w0 + ff) & 7])
                        v1 = plsc.load_gather(
                            stage_v,
                            [zeros + p, (w0 + 2 + ff) >> 3, (w0 + 2 + ff) & 7])
                        s = wx[j] * (wy[0] * v0 + wy[1] * v1)
                        acc[ff] = s if acc[ff] is None else acc[ff] + s
                accs.append(acc)
            for ff in range(F):
                o = jnp.maximum(accs[0][ff] * accs[1][ff], 1e-6)
                o = jnp.maximum(o * accs[2][ff], 1e-6)
                fcol = 2 * l + ff
                plsc.store_scatter(
                    out_v, [zeros + (fcol >> 3), zeros + (fcol & 7), col], o)
            return carry

        lax.fori_loop(0, G, grp, 0)

    def chunk_body(ch, carry):
        pt = base0 + ch * C
        for p, inp in enumerate((in0, in1, in2)):
            pltpu.sync_copy(inp.at[pl.ds(pt, C)], coords_v.at[p])
        for l in range(NST, NST + D):
            fire(l, l < NDENSE)
        for l in range(NST):
            interp_staged(l)
        for l in range(NST, NLVL):
            interp(l, l < NDENSE)
            if l + D < NLVL:
                fire(l + D, (l + D) < NDENSE)
        pltpu.sync_copy(out_v, out.at[:, base0 // 128 + ch])
        return carry

    lax.fori_loop(0, NCH, chunk_body, 0)


def kernel(input, table0, table1, table2):
    in0, in1, in2 = input[:, 0], input[:, 1], input[:, 2]

    def view(t):
        return t.reshape(NLVL, T // 128, 128, F).transpose(0, 1, 3, 2)

    t0, t1, t2, p0, p1, p2 = _build_relayout()(
        view(table0), view(table1), view(table2))
    out4 = _build_kernel()(in0, in1, in2, t0, t1, t2, p0, p1, p2)
    return out4.transpose(1, 3, 0, 2).reshape(N, 2 * NLVL)
